# Initial kernel scaffold; baseline (speedup 1.0000x reference)
import jax
import jax.numpy as jnp
from jax.experimental import pallas as pl


def kernel(ivectors, data):
    raise NotImplementedError("write your pallas kernel here")


# speedup vs baseline: 1.7862x; 1.7862x over previous
import functools

import jax
import jax.numpy as jnp
from jax import lax
from jax.experimental import pallas as pl
from jax.experimental.pallas import tpu as pltpu
from jax.experimental.pallas import tpu_sc as plsc

V = 1000001
D = 64
B = 16384 * 50
NC, NS = 2, 16
NW = NC * NS
IR = 128
G = 4
CH = G * IR
ROWS_PER_W = B // (NW * IR)
NCHUNK = ROWS_PER_W // G


def _fix_zero_rows(idx_v, rows_v):
    lane = lax.iota(jnp.int32, 16)

    def fixrow(r, carry):
        jj = jnp.full((16,), r // IR, jnp.int32)
        cc = jnp.full((16,), r % IR, jnp.int32)
        iv = plsc.load_gather(idx_v, [jj, cc])
        m = jnp.where(iv == 0, jnp.float32(0.0), jnp.float32(1.0))
        rr = jnp.full((16,), r, jnp.int32)
        for k in range(D // 16):
            col = lane + k * 16
            x = plsc.load_gather(rows_v, [rr, col])
            plsc.store_scatter(rows_v, [rr, col], x * m)
        return carry

    lax.fori_loop(0, CH, fixrow, 0)


def _make_kernel():
    mesh = plsc.VectorSubcoreMesh(core_axis_name="c", subcore_axis_name="s")

    @functools.partial(
        pl.kernel,
        mesh=mesh,
        compiler_params=pltpu.CompilerParams(
            needs_layout_passes=False, use_tc_tiling_on_sc=False
        ),
        out_type=jax.ShapeDtypeStruct((B, D), jnp.float32),
        scratch_types=[
            pltpu.VMEM((G, IR), jnp.int32),
            pltpu.VMEM((CH, D), jnp.float32),
            pltpu.SemaphoreType.DMA,
        ],
    )
    def gather_kernel(table_hbm, idx_hbm, out_hbm, idx_v, rows_v, sem):
        wid = lax.axis_index("s") * NC + lax.axis_index("c")
        row_base = wid * ROWS_PER_W
        idx_base = row_base * IR

        def chunk(g, carry):
            roff = row_base + g * G
            pltpu.sync_copy(idx_hbm.at[pl.ds(roff, G)], idx_v)
            copies = [
                pltpu.async_copy(
                    table_hbm.at[idx_v.at[j]],
                    rows_v.at[pl.ds(j * IR, IR)],
                    sem,
                )
                for j in range(G)
            ]
            mn = jnp.full((16,), 1, jnp.int32)
            for j in range(G):
                for l in range(IR // 16):
                    mn = jnp.minimum(mn, idx_v[j, pl.ds(l * 16, 16)])
            nzero = plsc.all_reduce_population_count(mn == 0)
            has_zero = nzero[0] > 0
            for c in copies:
                c.wait()
            pl.when(has_zero)(lambda: _fix_zero_rows(idx_v, rows_v))
            pltpu.sync_copy(rows_v, out_hbm.at[pl.ds(idx_base + g * CH, CH)])
            return carry

        lax.fori_loop(0, NCHUNK, chunk, 0)

    return gather_kernel


@functools.lru_cache(maxsize=1)
def _get_kernel():
    return _make_kernel()


def kernel(ivectors, data):
    idx = data.astype(jnp.int32).reshape(B // IR, IR)
    out = _get_kernel()(ivectors, idx)
    return out.reshape(data.shape[0], data.shape[1], D)


# speedup vs baseline: 1.8713x; 1.0476x over previous
import functools

import jax
import jax.numpy as jnp
from jax import lax
from jax.experimental import pallas as pl
from jax.experimental.pallas import tpu as pltpu
from jax.experimental.pallas import tpu_sc as plsc

V = 1000001
D = 64
B = 16384 * 50
NC, NS = 2, 16
NW = NC * NS
IR = 128
G = 5
CH = G * IR
ROWS_PER_W = B // (NW * IR)
NCHUNK = ROWS_PER_W // G


def _idx_splat16(idx_v, flat):
    row = jnp.full((16,), flat // IR, jnp.int32)
    col = jnp.full((16,), flat % IR, jnp.int32)
    return plsc.load_gather(idx_v, [row, col])


def _detect_zero(idx_v, flat_base):
    lane = lax.iota(jnp.int32, 16)
    mn = jnp.full((16,), 1, jnp.int32)
    for t in range(CH // 16):
        flat = flat_base + t * 16
        row = jnp.full((16,), flat // IR, jnp.int32)
        col = jnp.full((16,), flat % IR, jnp.int32) + lane
        mn = jnp.minimum(mn, plsc.load_gather(idx_v, [row, col]))
    nzero = plsc.all_reduce_population_count(mn == 0)
    return nzero[0] > 0


def _fix_zero_rows(idx_v, rows_v, flat_base):
    lane = lax.iota(jnp.int32, 16)

    def fixrow(r, carry):
        iv = _idx_splat16(idx_v, flat_base + r)
        m = jnp.where(iv == 0, jnp.float32(0.0), jnp.float32(1.0))
        rr = jnp.full((16,), r, jnp.int32)
        for k in range(D // 16):
            col = lane + k * 16
            x = plsc.load_gather(rows_v, [rr, col])
            plsc.store_scatter(rows_v, [rr, col], x * m)
        return carry

    lax.fori_loop(0, CH, fixrow, 0)


def _make_kernel():
    mesh = plsc.VectorSubcoreMesh(core_axis_name="c", subcore_axis_name="s")

    @functools.partial(
        pl.kernel,
        mesh=mesh,
        compiler_params=pltpu.CompilerParams(
            needs_layout_passes=False, use_tc_tiling_on_sc=False
        ),
        out_type=jax.ShapeDtypeStruct((B, D), jnp.float32),
        scratch_types=[
            pltpu.VMEM((ROWS_PER_W, IR), jnp.int32),
            pltpu.VMEM((CH, D), jnp.float32),
            pltpu.VMEM((CH, D), jnp.float32),
            pltpu.SemaphoreType.DMA,
            pltpu.SemaphoreType.DMA,
            pltpu.SemaphoreType.DMA,
            pltpu.SemaphoreType.DMA,
        ],
    )
    def gather_kernel(
        table_hbm, idx_hbm, out_hbm,
        idx_v, rows0, rows1, gsem0, gsem1, osem0, osem1,
    ):
        wid = lax.axis_index("s") * NC + lax.axis_index("c")
        row_base = wid * ROWS_PER_W
        idx_base = row_base * IR
        rows = (rows0, rows1)
        gsem = (gsem0, gsem1)
        osem = (osem0, osem1)

        pltpu.sync_copy(idx_hbm.at[pl.ds(row_base, ROWS_PER_W)], idx_v)

        def fire_gather(g, b):
            for j in range(G):
                pltpu.async_copy(
                    table_hbm.at[idx_v.at[g * G + j]],
                    rows[b].at[pl.ds(j * IR, IR)],
                    gsem[b],
                )

        def drain_gather(b):
            pltpu.make_async_copy(
                table_hbm.at[pl.ds(0, CH)], rows[b], gsem[b]
            ).wait()

        def fire_out(g, b):
            pltpu.async_copy(
                rows[b], out_hbm.at[pl.ds(idx_base + g * CH, CH)], osem[b]
            )

        def drain_out(b):
            pltpu.make_async_copy(
                rows[b], out_hbm.at[pl.ds(0, CH)], osem[b]
            ).wait()

        fire_gather(0, 0)

        def outer(k, carry):
            for b in range(2):
                g = k * 2 + b
                nb = 1 - b
                pl.when((g >= 1) & (g + 1 < NCHUNK))(lambda: drain_out(nb))
                pl.when(g + 1 < NCHUNK)(lambda: fire_gather(g + 1, nb))
                has_zero = _detect_zero(idx_v, g * CH)
                drain_gather(b)
                pl.when(has_zero)(
                    lambda: _fix_zero_rows(idx_v, rows[b], g * CH)
                )
                fire_out(g, b)
            return carry

        lax.fori_loop(0, NCHUNK // 2, outer, 0)
        drain_out(0)
        drain_out(1)

    return gather_kernel


@functools.lru_cache(maxsize=1)
def _get_kernel():
    return _make_kernel()


def kernel(ivectors, data):
    idx = data.astype(jnp.int32).reshape(B // IR, IR)
    out = _get_kernel()(ivectors, idx)
    return out.reshape(data.shape[0], data.shape[1], D)
